# baseline (speedup 1.0000x reference)
import functools

import jax
import jax.numpy as jnp
from jax import lax
from jax.experimental import pallas as pl
from jax.experimental.pallas import tpu as pltpu
from jax.experimental.pallas import tpu_sc as plsc

D = 64
NC, NS = 2, 16
NW = NC * NS
CHUNK = 320
QCHUNK = 512
QW = 16


def _widen_table(v_rows):
    blk = 4096

    def body(x_ref, o_ref):
        o_ref[...] = jnp.concatenate(
            [x_ref[...], jnp.zeros((blk, D), jnp.float32)], axis=1)

    return pl.pallas_call(
        body,
        grid=(pl.cdiv(v_rows, blk),),
        in_specs=[pl.BlockSpec((blk, D), lambda i: (i, 0))],
        out_specs=pl.BlockSpec((blk, 2 * D), lambda i: (i, 0)),
        out_shape=jax.ShapeDtypeStruct((v_rows, 2 * D), jnp.float32),
    )


def _sc_gather(n_tokens):
    per_w = n_tokens // NW
    n_chunks = per_w // CHUNK
    mesh = plsc.VectorSubcoreMesh(core_axis_name="c", subcore_axis_name="s")

    @functools.partial(
        pl.kernel,
        out_type=jax.ShapeDtypeStruct((n_tokens, 2 * D), jnp.float32),
        mesh=mesh,
        scratch_types=[
            pltpu.VMEM((CHUNK,), jnp.int32),
            pltpu.VMEM((CHUNK,), jnp.int32),
            pltpu.VMEM((CHUNK, 2 * D), jnp.float32),
            pltpu.VMEM((CHUNK, 2 * D), jnp.float32),
            pltpu.SemaphoreType.DMA,
        ],
    )
    def gather(fid_hbm, cid_hbm, fine_tab, coarse_tab, packed_out,
               fidx_v, cidx_v, packed_v, crows_v, sem):
        wid = lax.axis_index("s") * NC + lax.axis_index("c")
        base = wid * per_w

        def body(i, carry):
            off = pl.multiple_of(base + i * CHUNK, CHUNK)
            pltpu.sync_copy(fid_hbm.at[pl.ds(off, CHUNK)], fidx_v)
            pltpu.sync_copy(cid_hbm.at[pl.ds(off, CHUNK)], cidx_v)
            a = pltpu.async_copy(fine_tab.at[fidx_v], packed_v, sem)
            b = pltpu.async_copy(coarse_tab.at[cidx_v], crows_v, sem)
            a.wait()
            b.wait()

            def pack_body(t8, carry2):
                for t in range(8):
                    for j in range(D // 16):
                        s = pl.ds(D + j * 16, 16)
                        s0 = pl.ds(j * 16, 16)
                        packed_v[t8 * 8 + t, s] = crows_v[t8 * 8 + t, s0]
                return carry2

            lax.fori_loop(0, CHUNK // 8, pack_body, 0)
            pltpu.sync_copy(packed_v, packed_out.at[pl.ds(off, CHUNK)])
            return carry

        lax.fori_loop(0, n_chunks, body, 0)

    return gather


def _sc_freq(n_tokens):
    per_w = n_tokens // NW
    n_chunks = per_w // QCHUNK
    qrows_per_chunk = QCHUNK // 128
    mesh = plsc.VectorSubcoreMesh(core_axis_name="c", subcore_axis_name="s")

    @functools.partial(
        pl.kernel,
        out_type=jax.ShapeDtypeStruct((n_tokens,), jnp.float32),
        mesh=mesh,
        compiler_params=pltpu.CompilerParams(
            use_tc_tiling_on_sc=False, needs_layout_passes=False),
        scratch_types=[
            pltpu.VMEM((QCHUNK,), jnp.int32),
            pltpu.VMEM((QCHUNK,), jnp.int32),
            pltpu.VMEM((QCHUNK, QW), jnp.float32),
            pltpu.VMEM((QCHUNK,), jnp.float32),
            pltpu.SemaphoreType.DMA,
        ],
    )
    def freq_gather(fid_hbm, freq_tab, fw_out,
                    fidx_v, qidx_v, qrows_v, qout_v, sem):
        wid = lax.axis_index("s") * NC + lax.axis_index("c")
        base = wid * per_w

        def body(i, carry):
            off = pl.multiple_of(base + i * QCHUNK, QCHUNK)
            pltpu.sync_copy(fid_hbm.at[pl.ds(off, QCHUNK)], fidx_v)
            for k in range(QCHUNK // 16):
                s = pl.ds(k * 16, 16)
                qidx_v[s] = lax.shift_right_logical(fidx_v[s], 4)
            pltpu.async_copy(freq_tab.at[qidx_v], qrows_v, sem).wait()
            lane0 = lax.iota(jnp.int32, 16)
            for k in range(QCHUNK // 16):
                s = pl.ds(k * 16, 16)
                rows = lane0 + k * 16
                lanes = lax.bitwise_and(fidx_v[s], QW - 1)
                q = plsc.load_gather(qrows_v, [rows, lanes])
                qout_v[s] = 1.0 / (1.0 + jnp.exp(-q))
            pltpu.sync_copy(qout_v, fw_out.at[pl.ds(off, QCHUNK)])
            return carry

        lax.fori_loop(0, n_chunks, body, 0)

    return freq_gather


def _tc_mlp_body(packed_ref, fw_ref, w1fc_ref, w1q_ref,
                 b1_ref, w2_ref, b2_ref, fused_ref, gate_ref):
    bb, l_len, _ = fused_ref.shape
    x = packed_ref[...]
    fw = fw_ref[...]
    h = jnp.dot(x, w1fc_ref[...], preferred_element_type=jnp.float32)
    h += fw * w1q_ref[...] + b1_ref[...]
    h = jnp.maximum(h, 0.0)
    g = jnp.sum(h * w2_ref[...], axis=1, keepdims=True) + b2_ref[...]
    ag = jax.nn.sigmoid(g) * fw
    fine = x[:, :D]
    coarse = x[:, D:]
    fused = coarse + ag * (fine - coarse)
    fused_ref[...] = fused.reshape(bb, l_len, D)
    gate_ref[...] = ag.reshape(bb, l_len, 1)


def _tc_mlp(b_rows, l_len, bb):
    bn = bb * l_len
    grid = (b_rows // bb,)
    return pl.pallas_call(
        _tc_mlp_body,
        grid=grid,
        in_specs=[
            pl.BlockSpec((bn, 2 * D), lambda i: (i, 0)),
            pl.BlockSpec((bn, 1), lambda i: (i, 0)),
            pl.BlockSpec((2 * D, 32), lambda i: (0, 0)),
            pl.BlockSpec((1, 32), lambda i: (0, 0)),
            pl.BlockSpec((1, 32), lambda i: (0, 0)),
            pl.BlockSpec((1, 32), lambda i: (0, 0)),
            pl.BlockSpec((1, 1), lambda i: (0, 0)),
        ],
        out_specs=[
            pl.BlockSpec((bb, l_len, D), lambda i: (i, 0, 0)),
            pl.BlockSpec((bb, l_len, 1), lambda i: (i, 0, 0)),
        ],
        out_shape=[
            jax.ShapeDtypeStruct((b_rows, l_len, D), jnp.float32),
            jax.ShapeDtypeStruct((b_rows, l_len, 1), jnp.float32),
        ],
    )


def kernel(fine_ids, coarse_ids, fine_table, coarse_table, freq_table,
           W1, b1, W2, b2):
    B, L = fine_ids.shape
    n = B * L
    fid = fine_ids.reshape(n).astype(jnp.int32)
    cid = coarse_ids.reshape(n).astype(jnp.int32)

    v = freq_table.shape[0]
    pad = (-v) % QW
    freq16 = jnp.pad(freq_table.reshape(v), (0, pad)).reshape(-1, QW)
    fine_w = _widen_table(fine_table.shape[0])(fine_table)
    coarse_w = _widen_table(coarse_table.shape[0])(coarse_table)

    packed = _sc_gather(n)(fid, cid, fine_w, coarse_w)
    fw_col = _sc_freq(n)(fid, freq16).reshape(n, 1)

    w1fc = W1[:2 * D]
    w1q = W1[2 * D:]
    fused, gate = _tc_mlp(B, L, 64)(
        packed, fw_col, w1fc, w1q,
        b1.reshape(1, 32), W2.reshape(1, 32), b2.reshape(1, 1))
    return fused, gate


# speedup vs baseline: 1.0358x; 1.0358x over previous
import functools

import jax
import jax.numpy as jnp
from jax import lax
from jax.experimental import pallas as pl
from jax.experimental.pallas import tpu as pltpu
from jax.experimental.pallas import tpu_sc as plsc

D = 64
NC, NS = 2, 16
NW = NC * NS
CHUNK = 200
QCHUNK = 1600
QW = 16


def _widen_table(v_rows):
    blk = 4096

    def body(x_ref, o_ref):
        o_ref[...] = jnp.concatenate(
            [x_ref[...], jnp.zeros((blk, D), jnp.float32)], axis=1)

    return pl.pallas_call(
        body,
        grid=(pl.cdiv(v_rows, blk),),
        in_specs=[pl.BlockSpec((blk, D), lambda i: (i, 0))],
        out_specs=pl.BlockSpec((blk, 2 * D), lambda i: (i, 0)),
        out_shape=jax.ShapeDtypeStruct((v_rows, 2 * D), jnp.float32),
    )


def _sc_gather(n_tokens):
    per_w = n_tokens // NW
    n_chunks = per_w // CHUNK
    mesh = plsc.VectorSubcoreMesh(core_axis_name="c", subcore_axis_name="s")

    @functools.partial(
        pl.kernel,
        out_type=jax.ShapeDtypeStruct((n_tokens, 2 * D), jnp.float32),
        mesh=mesh,
        scratch_types=[
            pltpu.VMEM((CHUNK,), jnp.int32),
            pltpu.VMEM((CHUNK,), jnp.int32),
            pltpu.VMEM((CHUNK,), jnp.int32),
            pltpu.VMEM((CHUNK,), jnp.int32),
            pltpu.VMEM((CHUNK, 2 * D), jnp.float32),
            pltpu.VMEM((CHUNK, 2 * D), jnp.float32),
            pltpu.VMEM((CHUNK, 2 * D), jnp.float32),
            pltpu.VMEM((CHUNK, 2 * D), jnp.float32),
            pltpu.SemaphoreType.DMA,
            pltpu.SemaphoreType.DMA,
            pltpu.SemaphoreType.DMA,
            pltpu.SemaphoreType.DMA,
        ],
    )
    def gather(fid_hbm, cid_hbm, fine_tab, coarse_tab, packed_out,
               fidx0, fidx1, cidx0, cidx1, packed0, packed1, crows0, crows1,
               gsem0, gsem1, wsem0, wsem1):
        wid = lax.axis_index("s") * NC + lax.axis_index("c")
        base = wid * per_w
        fidxs = (fidx0, fidx1)
        cidxs = (cidx0, cidx1)
        packs = (packed0, packed1)
        crows = (crows0, crows1)
        gsems = (gsem0, gsem1)
        wsems = (wsem0, wsem1)

        def fetch(g, b):
            off = pl.multiple_of(base + g * CHUNK, CHUNK)
            pltpu.sync_copy(fid_hbm.at[pl.ds(off, CHUNK)], fidxs[b])
            pltpu.sync_copy(cid_hbm.at[pl.ds(off, CHUNK)], cidxs[b])
            pltpu.async_copy(fine_tab.at[fidxs[b]], packs[b], gsems[b])
            pltpu.async_copy(coarse_tab.at[cidxs[b]], crows[b], gsems[b])

        def wait_gathers(b):
            pltpu.make_async_copy(fine_tab.at[fidxs[b]], packs[b],
                                  gsems[b]).wait()
            pltpu.make_async_copy(coarse_tab.at[cidxs[b]], crows[b],
                                  gsems[b]).wait()

        def wait_wb(g, b):
            off = pl.multiple_of(base + g * CHUNK, CHUNK)
            pltpu.make_async_copy(packs[b],
                                  packed_out.at[pl.ds(off, CHUNK)],
                                  wsems[b]).wait()

        def consume(g, b):
            def pack_body(t8, carry2):
                for t in range(8):
                    for j in range(D // 16):
                        s = pl.ds(D + j * 16, 16)
                        s0 = pl.ds(j * 16, 16)
                        packs[b][t8 * 8 + t, s] = crows[b][t8 * 8 + t, s0]
                return carry2

            lax.fori_loop(0, CHUNK // 8, pack_body, 0)
            off = pl.multiple_of(base + g * CHUNK, CHUNK)
            pltpu.async_copy(packs[b],
                             packed_out.at[pl.ds(off, CHUNK)], wsems[b])

        fetch(0, 0)
        fetch(1, 1)

        def body(gp, carry):
            g0 = gp * 2
            wait_gathers(0)
            consume(g0, 0)
            wait_gathers(1)
            consume(g0 + 1, 1)

            @pl.when(g0 + 2 < n_chunks)
            def _():
                wait_wb(g0, 0)
                fetch(g0 + 2, 0)
                wait_wb(g0 + 1, 1)
                fetch(g0 + 3, 1)

            return carry

        lax.fori_loop(0, n_chunks // 2, body, 0)
        wait_wb(n_chunks - 2, 0)
        wait_wb(n_chunks - 1, 1)

    return gather


def _sc_freq(n_tokens):
    per_w = n_tokens // NW
    n_chunks = per_w // QCHUNK
    qrows_per_chunk = QCHUNK // 128
    mesh = plsc.VectorSubcoreMesh(core_axis_name="c", subcore_axis_name="s")

    @functools.partial(
        pl.kernel,
        out_type=jax.ShapeDtypeStruct((n_tokens,), jnp.float32),
        mesh=mesh,
        compiler_params=pltpu.CompilerParams(
            use_tc_tiling_on_sc=False, needs_layout_passes=False),
        scratch_types=[
            pltpu.VMEM((QCHUNK,), jnp.int32),
            pltpu.VMEM((QCHUNK,), jnp.int32),
            pltpu.VMEM((QCHUNK, QW), jnp.float32),
            pltpu.VMEM((QCHUNK,), jnp.float32),
            pltpu.SemaphoreType.DMA,
        ],
    )
    def freq_gather(fid_hbm, freq_tab, fw_out,
                    fidx_v, qidx_v, qrows_v, qout_v, sem):
        wid = lax.axis_index("s") * NC + lax.axis_index("c")
        base = wid * per_w

        def body(i, carry):
            off = pl.multiple_of(base + i * QCHUNK, QCHUNK)
            pltpu.sync_copy(fid_hbm.at[pl.ds(off, QCHUNK)], fidx_v)
            for k in range(QCHUNK // 16):
                s = pl.ds(k * 16, 16)
                qidx_v[s] = lax.shift_right_logical(fidx_v[s], 4)
            pltpu.async_copy(freq_tab.at[qidx_v], qrows_v, sem).wait()
            lane0 = lax.iota(jnp.int32, 16)
            for k in range(QCHUNK // 16):
                s = pl.ds(k * 16, 16)
                rows = lane0 + k * 16
                lanes = lax.bitwise_and(fidx_v[s], QW - 1)
                q = plsc.load_gather(qrows_v, [rows, lanes])
                qout_v[s] = 1.0 / (1.0 + jnp.exp(-q))
            pltpu.sync_copy(qout_v, fw_out.at[pl.ds(off, QCHUNK)])
            return carry

        lax.fori_loop(0, n_chunks, body, 0)

    return freq_gather


def _tc_mlp_body(packed_ref, fw_ref, w1fc_ref, w1q_ref,
                 b1_ref, w2_ref, b2_ref, fused_ref, gate_ref):
    bb, l_len, _ = fused_ref.shape
    x = packed_ref[...]
    fw = fw_ref[...]
    h = jnp.dot(x, w1fc_ref[...], preferred_element_type=jnp.float32)
    h += fw * w1q_ref[...] + b1_ref[...]
    h = jnp.maximum(h, 0.0)
    g = jnp.sum(h * w2_ref[...], axis=1, keepdims=True) + b2_ref[...]
    ag = jax.nn.sigmoid(g) * fw
    fine = x[:, :D]
    coarse = x[:, D:]
    fused = coarse + ag * (fine - coarse)
    fused_ref[...] = fused.reshape(bb, l_len, D)
    gate_ref[...] = ag.reshape(bb, l_len, 1)


def _tc_mlp(b_rows, l_len, bb):
    bn = bb * l_len
    grid = (b_rows // bb,)
    return pl.pallas_call(
        _tc_mlp_body,
        grid=grid,
        in_specs=[
            pl.BlockSpec((bn, 2 * D), lambda i: (i, 0)),
            pl.BlockSpec((bn, 1), lambda i: (i, 0)),
            pl.BlockSpec((2 * D, 32), lambda i: (0, 0)),
            pl.BlockSpec((1, 32), lambda i: (0, 0)),
            pl.BlockSpec((1, 32), lambda i: (0, 0)),
            pl.BlockSpec((1, 32), lambda i: (0, 0)),
            pl.BlockSpec((1, 1), lambda i: (0, 0)),
        ],
        out_specs=[
            pl.BlockSpec((bb, l_len, D), lambda i: (i, 0, 0)),
            pl.BlockSpec((bb, l_len, 1), lambda i: (i, 0, 0)),
        ],
        out_shape=[
            jax.ShapeDtypeStruct((b_rows, l_len, D), jnp.float32),
            jax.ShapeDtypeStruct((b_rows, l_len, 1), jnp.float32),
        ],
    )


def kernel(fine_ids, coarse_ids, fine_table, coarse_table, freq_table,
           W1, b1, W2, b2):
    B, L = fine_ids.shape
    n = B * L
    fid = fine_ids.reshape(n).astype(jnp.int32)
    cid = coarse_ids.reshape(n).astype(jnp.int32)

    v = freq_table.shape[0]
    pad = (-v) % QW
    freq16 = jnp.pad(freq_table.reshape(v), (0, pad)).reshape(-1, QW)
    fine_w = _widen_table(fine_table.shape[0])(fine_table)
    coarse_w = _widen_table(coarse_table.shape[0])(coarse_table)

    packed = _sc_gather(n)(fid, cid, fine_w, coarse_w)
    fw_col = _sc_freq(n)(fid, freq16).reshape(n, 1)

    w1fc = W1[:2 * D]
    w1q = W1[2 * D:]
    fused, gate = _tc_mlp(B, L, 64)(
        packed, fw_col, w1fc, w1q,
        b1.reshape(1, 32), W2.reshape(1, 32), b2.reshape(1, 1))
    return fused, gate
